# Initial kernel scaffold; baseline (speedup 1.0000x reference)
import jax
import jax.numpy as jnp
from jax.experimental import pallas as pl


def kernel(x, W1l, b1l, W1r, b1r, att1, bias1, W2l, b2l, W2r, b2r, att2, bias2):
    raise NotImplementedError("write your pallas kernel here")


# speedup vs baseline: 54.1507x; 54.1507x over previous
import functools

import jax
import jax.numpy as jnp
from jax.experimental import pallas as pl

N = 256
HEADS = 4
C1 = 32
OUT = 128
NEG = 0.2
CH1 = 64
CH2 = 32


def _lrelu(v):
    return jnp.where(v > 0, v, NEG * v)


def _attend(xl_h, xr_h, a):
    t = _lrelu(xl_h[None, :, :] + xr_h[:, None, :])
    logits = jnp.sum(t * a[None, None, :], axis=-1)
    m = jnp.max(logits, axis=1, keepdims=True)
    e = jnp.exp(logits - m)
    alpha = e / jnp.sum(e, axis=1, keepdims=True)
    return jnp.dot(alpha, xl_h, preferred_element_type=jnp.float32)


def _layer1_body(x_ref, Wl_ref, bl_ref, Wr_ref, br_ref, att_ref, bias_ref,
                 out_ref):
    f32 = jnp.float32
    d0 = pl.program_id(0) * CH1
    x = x_ref[...]
    xl = jnp.dot(x, Wl_ref[...], preferred_element_type=f32) + bl_ref[...]
    xc = x_ref[pl.ds(d0, CH1), :]
    xr = jnp.dot(xc, Wr_ref[...], preferred_element_type=f32) + br_ref[...]
    att = att_ref[...]
    cols = []
    for h in range(HEADS):
        cols.append(_attend(xl[:, h * C1:(h + 1) * C1],
                            xr[:, h * C1:(h + 1) * C1], att[h]))
    hfeat = jnp.concatenate(cols, axis=1) + bias_ref[...]
    out_ref[...] = jnp.maximum(hfeat, 0.0)


def _layer2_body(h_ref, Wl_ref, bl_ref, Wr_ref, br_ref, att_ref, bias_ref,
                 out_ref):
    f32 = jnp.float32
    d0 = pl.program_id(0) * CH2
    hfull = h_ref[...]
    xl = jnp.dot(hfull, Wl_ref[...], preferred_element_type=f32) + bl_ref[...]
    hc = h_ref[pl.ds(d0, CH2), :]
    xr = jnp.dot(hc, Wr_ref[...], preferred_element_type=f32) + br_ref[...]
    att = att_ref[...]
    acc = jnp.zeros((CH2, OUT), f32)
    for h in range(HEADS):
        acc = acc + _attend(xl[:, h * OUT:(h + 1) * OUT],
                            xr[:, h * OUT:(h + 1) * OUT], att[h])
    out_ref[...] = acc * (1.0 / HEADS) + bias_ref[...]


def _full(shape):
    return pl.BlockSpec(shape, lambda i: (0,) * len(shape))


def kernel(x, W1l, b1l, W1r, b1r, att1, bias1, W2l, b2l, W2r, b2r, att2,
           bias2):
    f32 = jnp.float32
    hfeat = pl.pallas_call(
        _layer1_body,
        grid=(N // CH1,),
        in_specs=[
            _full((N, x.shape[1])), _full(W1l.shape), _full((1, HEADS * C1)),
            _full(W1r.shape), _full((1, HEADS * C1)), _full(att1.shape),
            _full((1, HEADS * C1)),
        ],
        out_specs=pl.BlockSpec((CH1, HEADS * C1), lambda i: (i, 0)),
        out_shape=jax.ShapeDtypeStruct((N, HEADS * C1), f32),
    )(x, W1l, b1l.reshape(1, -1), W1r, b1r.reshape(1, -1), att1,
      bias1.reshape(1, -1))

    out = pl.pallas_call(
        _layer2_body,
        grid=(N // CH2,),
        in_specs=[
            _full((N, HEADS * C1)), _full(W2l.shape),
            _full((1, HEADS * OUT)), _full(W2r.shape),
            _full((1, HEADS * OUT)), _full(att2.shape), _full((1, OUT)),
        ],
        out_specs=pl.BlockSpec((CH2, OUT), lambda i: (i, 0)),
        out_shape=jax.ShapeDtypeStruct((N, OUT), f32),
    )(hfeat, W2l, b2l.reshape(1, -1), W2r, b2r.reshape(1, -1), att2,
      bias2.reshape(1, -1))
    return out


# speedup vs baseline: 54.1877x; 1.0007x over previous
import jax
import jax.numpy as jnp
from jax.experimental import pallas as pl
from jax.experimental.pallas import tpu as pltpu

N = 256
HEADS = 4
C1 = 32
OUT = 128
NEG = 0.2
CH1 = 64
CH2 = 32


def _lrelu(v):
    return jnp.where(v > 0, v, NEG * v)


def _attend(xl_h, xr_h, a):
    t = _lrelu(xl_h[None, :, :] + xr_h[:, None, :])
    logits = jnp.sum(t * a[None, None, :], axis=-1)
    m = jnp.max(logits, axis=1, keepdims=True)
    e = jnp.exp(logits - m)
    alpha = e / jnp.sum(e, axis=1, keepdims=True)
    return jnp.dot(alpha, xl_h, preferred_element_type=jnp.float32)


def _layer1_body(x_ref, Wl_ref, bl_ref, Wr_ref, br_ref, att_ref, bias_ref,
                 out_ref, xl_scr):
    f32 = jnp.float32
    d0 = pl.program_id(0) * CH1

    @pl.when(pl.program_id(0) == 0)
    def _():
        x = x_ref[...]
        xl_scr[...] = (jnp.dot(x, Wl_ref[...], preferred_element_type=f32)
                       + bl_ref[...])

    xl = xl_scr[...]
    xc = x_ref[pl.ds(d0, CH1), :]
    xr = jnp.dot(xc, Wr_ref[...], preferred_element_type=f32) + br_ref[...]
    att = att_ref[...]
    cols = []
    for h in range(HEADS):
        cols.append(_attend(xl[:, h * C1:(h + 1) * C1],
                            xr[:, h * C1:(h + 1) * C1], att[h]))
    hfeat = jnp.concatenate(cols, axis=1) + bias_ref[...]
    out_ref[...] = jnp.maximum(hfeat, 0.0)


def _layer2_body(h_ref, Wl_ref, bl_ref, Wr_ref, br_ref, att_ref, bias_ref,
                 out_ref, xl_scr):
    f32 = jnp.float32
    d0 = pl.program_id(0) * CH2

    @pl.when(pl.program_id(0) == 0)
    def _():
        hfull = h_ref[...]
        xl_scr[...] = (jnp.dot(hfull, Wl_ref[...],
                               preferred_element_type=f32) + bl_ref[...])

    xl = xl_scr[...]
    hc = h_ref[pl.ds(d0, CH2), :]
    xr = jnp.dot(hc, Wr_ref[...], preferred_element_type=f32) + br_ref[...]
    att = att_ref[...]
    acc = jnp.zeros((CH2, OUT), f32)
    for h in range(HEADS):
        acc = acc + _attend(xl[:, h * OUT:(h + 1) * OUT],
                            xr[:, h * OUT:(h + 1) * OUT], att[h])
    out_ref[...] = acc * (1.0 / HEADS) + bias_ref[...]


def _full(shape):
    return pl.BlockSpec(shape, lambda i: (0,) * len(shape))


def kernel(x, W1l, b1l, W1r, b1r, att1, bias1, W2l, b2l, W2r, b2r, att2,
           bias2):
    f32 = jnp.float32
    hfeat = pl.pallas_call(
        _layer1_body,
        grid=(N // CH1,),
        in_specs=[
            _full((N, x.shape[1])), _full(W1l.shape), _full((1, HEADS * C1)),
            _full(W1r.shape), _full((1, HEADS * C1)), _full(att1.shape),
            _full((1, HEADS * C1)),
        ],
        out_specs=pl.BlockSpec((CH1, HEADS * C1), lambda i: (i, 0)),
        out_shape=jax.ShapeDtypeStruct((N, HEADS * C1), f32),
        scratch_shapes=[pltpu.VMEM((N, HEADS * C1), f32)],
    )(x, W1l, b1l.reshape(1, -1), W1r, b1r.reshape(1, -1), att1,
      bias1.reshape(1, -1))

    out = pl.pallas_call(
        _layer2_body,
        grid=(N // CH2,),
        in_specs=[
            _full((N, HEADS * C1)), _full(W2l.shape),
            _full((1, HEADS * OUT)), _full(W2r.shape),
            _full((1, HEADS * OUT)), _full(att2.shape), _full((1, OUT)),
        ],
        out_specs=pl.BlockSpec((CH2, OUT), lambda i: (i, 0)),
        out_shape=jax.ShapeDtypeStruct((N, OUT), f32),
        scratch_shapes=[pltpu.VMEM((N, HEADS * OUT), f32)],
    )(hfeat, W2l, b2l.reshape(1, -1), W2r, b2r.reshape(1, -1), att2,
      bias2.reshape(1, -1))
    return out


# speedup vs baseline: 54.4220x; 1.0043x over previous
import jax
import jax.numpy as jnp
from jax.experimental import pallas as pl
from jax.experimental.pallas import tpu as pltpu

N = 256
HEADS = 4
C1 = 32
OUT = 128
NEG = 0.2
CH1 = 64
CH2 = 32


def _lrelu(v):
    return jnp.maximum(v, NEG * v)


def _attend(xl_h, xr_h, a):
    t = _lrelu(xl_h[None, :, :] + xr_h[:, None, :])
    logits = jnp.sum(t * a[None, None, :], axis=-1)
    m = jnp.max(logits, axis=1, keepdims=True)
    e = jnp.exp(logits - m)
    alpha = e / jnp.sum(e, axis=1, keepdims=True)
    return jnp.dot(alpha, xl_h, preferred_element_type=jnp.float32)


def _layer1_body(x_ref, Wl_ref, bl_ref, Wr_ref, br_ref, att_ref, bias_ref,
                 out_ref):
    f32 = jnp.float32
    d0 = pl.program_id(0) * CH1
    x = x_ref[...]
    xl = jnp.dot(x, Wl_ref[...], preferred_element_type=f32) + bl_ref[...]
    xc = x_ref[pl.ds(d0, CH1), :]
    xr = jnp.dot(xc, Wr_ref[...], preferred_element_type=f32) + br_ref[...]
    att = att_ref[...]
    cols = []
    for h in range(HEADS):
        cols.append(_attend(xl[:, h * C1:(h + 1) * C1],
                            xr[:, h * C1:(h + 1) * C1], att[h]))
    hfeat = jnp.concatenate(cols, axis=1) + bias_ref[...]
    out_ref[...] = jnp.maximum(hfeat, 0.0)


def _layer2_body(h_ref, Wl_ref, bl_ref, Wr_ref, br_ref, att_ref, bias_ref,
                 out_ref):
    f32 = jnp.float32
    d0 = pl.program_id(0) * CH2
    hfull = h_ref[...]
    xl = jnp.dot(hfull, Wl_ref[...], preferred_element_type=f32) + bl_ref[...]
    hc = h_ref[pl.ds(d0, CH2), :]
    xr = jnp.dot(hc, Wr_ref[...], preferred_element_type=f32) + br_ref[...]
    att = att_ref[...]
    acc = jnp.zeros((CH2, OUT), f32)
    for h in range(HEADS):
        acc = acc + _attend(xl[:, h * OUT:(h + 1) * OUT],
                            xr[:, h * OUT:(h + 1) * OUT], att[h])
    out_ref[...] = acc * (1.0 / HEADS) + bias_ref[...]


def _full(shape):
    return pl.BlockSpec(shape, lambda i: (0,) * len(shape))


def kernel(x, W1l, b1l, W1r, b1r, att1, bias1, W2l, b2l, W2r, b2r, att2,
           bias2):
    f32 = jnp.float32
    hfeat = pl.pallas_call(
        _layer1_body,
        grid=(N // CH1,),
        in_specs=[
            _full((N, x.shape[1])), _full(W1l.shape), _full((1, HEADS * C1)),
            _full(W1r.shape), _full((1, HEADS * C1)), _full(att1.shape),
            _full((1, HEADS * C1)),
        ],
        out_specs=pl.BlockSpec((CH1, HEADS * C1), lambda i: (i, 0)),
        out_shape=jax.ShapeDtypeStruct((N, HEADS * C1), f32),
        compiler_params=pltpu.CompilerParams(
            dimension_semantics=("parallel",)),
    )(x, W1l, b1l.reshape(1, -1), W1r, b1r.reshape(1, -1), att1,
      bias1.reshape(1, -1))

    out = pl.pallas_call(
        _layer2_body,
        grid=(N // CH2,),
        in_specs=[
            _full((N, HEADS * C1)), _full(W2l.shape),
            _full((1, HEADS * OUT)), _full(W2r.shape),
            _full((1, HEADS * OUT)), _full(att2.shape), _full((1, OUT)),
        ],
        out_specs=pl.BlockSpec((CH2, OUT), lambda i: (i, 0)),
        out_shape=jax.ShapeDtypeStruct((N, OUT), f32),
        compiler_params=pltpu.CompilerParams(
            dimension_semantics=("parallel",)),
    )(hfeat, W2l, b2l.reshape(1, -1), W2r, b2r.reshape(1, -1), att2,
      bias2.reshape(1, -1))
    return out


# speedup vs baseline: 172.6433x; 3.1723x over previous
import jax
import jax.numpy as jnp
import numpy as np
from jax.experimental import pallas as pl
from jax.experimental.pallas import tpu as pltpu

N = 256
HEADS = 4
C1 = 32
OUT = 128
CH1 = 128
CH2 = 32
SIGNBIT = np.int32(-2147483648)
ABSMASK = np.int32(2147483647)


def _attend(xlT_s_h, xr_s_h, smask_h, xl_h, a06_col, logits_lin, f32):
    w = xlT_s_h[None, :, :] + xr_s_h[:, :, None]
    wi = jax.lax.bitcast_convert_type(w, jnp.int32)
    wi = jnp.bitwise_and(wi, ABSMASK)
    wi = jnp.bitwise_xor(wi, smask_h[None, :, :])
    ws = jax.lax.bitcast_convert_type(wi, f32)
    logits = logits_lin + jnp.sum(ws, axis=1)
    m = jnp.max(logits, axis=1, keepdims=True)
    e = jnp.exp(logits - m)
    alpha = e / jnp.sum(e, axis=1, keepdims=True)
    return jnp.dot(alpha, xl_h, preferred_element_type=f32)


def _layer_body(cdim, chunk, x_ref, xT_ref, Wl_ref, WlT_ref, blr_ref,
                blc_ref, Wr_ref, br_ref, att_ref, aval_ref, smask_ref,
                *out_and_scr):
    f32 = jnp.float32
    d0 = pl.program_id(0) * chunk
    x = x_ref[...]
    xl = jnp.dot(x, Wl_ref[...], preferred_element_type=f32) + blr_ref[...]
    xlT = (jnp.dot(WlT_ref[...], xT_ref[...], preferred_element_type=f32)
           + blc_ref[...])
    xc = x_ref[pl.ds(d0, chunk), :]
    xr = jnp.dot(xc, Wr_ref[...], preferred_element_type=f32) + br_ref[...]
    aval = aval_ref[...]
    smask = smask_ref[...]
    att = att_ref[...]
    xlT_s = xlT * aval
    xr_s = xr * aval[:, 0][None, :]
    cols = []
    for h in range(HEADS):
        sl = slice(h * cdim, (h + 1) * cdim)
        a06_col = 0.6 * att[h][:, None]
        A_row = jnp.sum(xlT[sl, :] * a06_col, axis=0, keepdims=True)
        B_col = jnp.sum(xr[:, sl] * a06_col[:, 0][None, :], axis=1,
                        keepdims=True)
        logits_lin = A_row + B_col
        cols.append(_attend(xlT_s[sl, :], xr_s[:, sl], smask[sl, :],
                            xl[:, sl], a06_col, logits_lin, f32))
    return cols, out_and_scr


def _layer1_body(x_ref, xT_ref, Wl_ref, WlT_ref, blr_ref, blc_ref, Wr_ref,
                 br_ref, att_ref, aval_ref, smask_ref, bias_ref, out_ref,
                 outT_ref):
    cols, _ = _layer_body(C1, CH1, x_ref, xT_ref, Wl_ref, WlT_ref, blr_ref,
                          blc_ref, Wr_ref, br_ref, att_ref, aval_ref,
                          smask_ref)
    hfeat = jnp.concatenate(cols, axis=1) + bias_ref[...]
    hfeat = jnp.maximum(hfeat, 0.0)
    out_ref[...] = hfeat
    outT_ref[...] = hfeat.T


def _layer2_body(x_ref, xT_ref, Wl_ref, WlT_ref, blr_ref, blc_ref, Wr_ref,
                 br_ref, att_ref, aval_ref, smask_ref, bias_ref, out_ref):
    cols, _ = _layer_body(OUT, CH2, x_ref, xT_ref, Wl_ref, WlT_ref, blr_ref,
                          blc_ref, Wr_ref, br_ref, att_ref, aval_ref,
                          smask_ref)
    acc = cols[0] + cols[1] + cols[2] + cols[3]
    out_ref[...] = acc * (1.0 / HEADS) + bias_ref[...]


def _full(shape):
    return pl.BlockSpec(shape, lambda i: (0,) * len(shape))


def _sign_inputs(att):
    a = att.reshape(-1, 1)
    aval = 0.4 * jnp.abs(a)
    smask = jnp.where(a < 0, SIGNBIT, jnp.int32(0))
    return aval, smask


def kernel(x, W1l, b1l, W1r, b1r, att1, bias1, W2l, b2l, W2r, b2r, att2,
           bias2):
    f32 = jnp.float32
    aval1, smask1 = _sign_inputs(att1)
    aval2, smask2 = _sign_inputs(att2)
    hid = HEADS * C1

    hfeat, hT = pl.pallas_call(
        _layer1_body,
        grid=(N // CH1,),
        in_specs=[
            _full((N, x.shape[1])), _full((x.shape[1], N)),
            _full(W1l.shape), _full((hid, x.shape[1])), _full((1, hid)),
            _full((hid, 1)), _full(W1r.shape), _full((1, hid)),
            _full(att1.shape), _full((hid, 1)), _full((hid, 1)),
            _full((1, hid)),
        ],
        out_specs=[pl.BlockSpec((CH1, hid), lambda i: (i, 0)),
                   pl.BlockSpec((hid, CH1), lambda i: (0, i))],
        out_shape=[jax.ShapeDtypeStruct((N, hid), f32),
                   jax.ShapeDtypeStruct((hid, N), f32)],
    )(x, x.T, W1l, W1l.T, b1l.reshape(1, -1), b1l.reshape(-1, 1), W1r,
      b1r.reshape(1, -1), att1, aval1, smask1, bias1.reshape(1, -1))

    wide = HEADS * OUT
    out = pl.pallas_call(
        _layer2_body,
        grid=(N // CH2,),
        in_specs=[
            _full((N, hid)), _full((hid, N)),
            _full(W2l.shape), _full((wide, hid)), _full((1, wide)),
            _full((wide, 1)), _full(W2r.shape), _full((1, wide)),
            _full(att2.shape), _full((wide, 1)), _full((wide, 1)),
            _full((1, OUT)),
        ],
        out_specs=pl.BlockSpec((CH2, OUT), lambda i: (i, 0)),
        out_shape=jax.ShapeDtypeStruct((N, OUT), f32),
    )(hfeat, hT, W2l, W2l.T, b2l.reshape(1, -1), b2l.reshape(-1, 1), W2r,
      b2r.reshape(1, -1), att2, aval2, smask2, bias2.reshape(1, -1))
    return out


# speedup vs baseline: 182.1992x; 1.0554x over previous
import jax
import jax.numpy as jnp
import numpy as np
from jax.experimental import pallas as pl
from jax.experimental.pallas import tpu as pltpu

N = 256
HEADS = 4
C1 = 32
OUT = 128
CH1 = 128
CH2 = 64
SIGNBIT = np.int32(-2147483648)
ABSMASK = np.int32(2147483647)


def _attend(xlT_s_h, xr_s_h, smask_h, xl_h, a06_col, logits_lin, f32):
    w = xlT_s_h[None, :, :] + xr_s_h[:, :, None]
    wi = jax.lax.bitcast_convert_type(w, jnp.int32)
    wi = jnp.bitwise_and(wi, ABSMASK)
    wi = jnp.bitwise_xor(wi, smask_h[None, :, :])
    ws = jax.lax.bitcast_convert_type(wi, f32)
    logits = logits_lin + jnp.sum(ws, axis=1)
    m = jnp.max(logits, axis=1, keepdims=True)
    e = jnp.exp(logits - m)
    alpha = e / jnp.sum(e, axis=1, keepdims=True)
    return jnp.dot(alpha, xl_h, preferred_element_type=f32)


def _layer_body(cdim, chunk, x_ref, xT_ref, Wl_ref, WlT_ref, blr_ref,
                blc_ref, Wr_ref, br_ref, att_ref, aval_ref, smask_ref,
                *out_and_scr):
    f32 = jnp.float32
    d0 = pl.program_id(0) * chunk
    x = x_ref[...]
    xl = jnp.dot(x, Wl_ref[...], preferred_element_type=f32) + blr_ref[...]
    xlT = (jnp.dot(WlT_ref[...], xT_ref[...], preferred_element_type=f32)
           + blc_ref[...])
    xc = x_ref[pl.ds(d0, chunk), :]
    xr = jnp.dot(xc, Wr_ref[...], preferred_element_type=f32) + br_ref[...]
    aval = aval_ref[...]
    smask = smask_ref[...]
    att = att_ref[...]
    xlT_s = xlT * aval
    xr_s = xr * aval[:, 0][None, :]
    cols = []
    for h in range(HEADS):
        sl = slice(h * cdim, (h + 1) * cdim)
        a06_col = 0.6 * att[h][:, None]
        A_row = jnp.sum(xlT[sl, :] * a06_col, axis=0, keepdims=True)
        B_col = jnp.sum(xr[:, sl] * a06_col[:, 0][None, :], axis=1,
                        keepdims=True)
        logits_lin = A_row + B_col
        cols.append(_attend(xlT_s[sl, :], xr_s[:, sl], smask[sl, :],
                            xl[:, sl], a06_col, logits_lin, f32))
    return cols, out_and_scr


def _layer1_body(x_ref, xT_ref, Wl_ref, WlT_ref, blr_ref, blc_ref, Wr_ref,
                 br_ref, att_ref, aval_ref, smask_ref, bias_ref, out_ref,
                 outT_ref):
    cols, _ = _layer_body(C1, CH1, x_ref, xT_ref, Wl_ref, WlT_ref, blr_ref,
                          blc_ref, Wr_ref, br_ref, att_ref, aval_ref,
                          smask_ref)
    hfeat = jnp.concatenate(cols, axis=1) + bias_ref[...]
    hfeat = jnp.maximum(hfeat, 0.0)
    out_ref[...] = hfeat
    outT_ref[...] = hfeat.T


def _layer2_body(x_ref, xT_ref, Wl_ref, WlT_ref, blr_ref, blc_ref, Wr_ref,
                 br_ref, att_ref, aval_ref, smask_ref, bias_ref, out_ref):
    cols, _ = _layer_body(OUT, CH2, x_ref, xT_ref, Wl_ref, WlT_ref, blr_ref,
                          blc_ref, Wr_ref, br_ref, att_ref, aval_ref,
                          smask_ref)
    acc = cols[0] + cols[1] + cols[2] + cols[3]
    out_ref[...] = acc * (1.0 / HEADS) + bias_ref[...]


def _full(shape):
    return pl.BlockSpec(shape, lambda i: (0,) * len(shape))


def _sign_inputs(att):
    a = att.reshape(-1, 1)
    aval = 0.4 * jnp.abs(a)
    smask = jnp.where(a < 0, SIGNBIT, jnp.int32(0))
    return aval, smask


def kernel(x, W1l, b1l, W1r, b1r, att1, bias1, W2l, b2l, W2r, b2r, att2,
           bias2):
    f32 = jnp.float32
    aval1, smask1 = _sign_inputs(att1)
    aval2, smask2 = _sign_inputs(att2)
    hid = HEADS * C1

    hfeat, hT = pl.pallas_call(
        _layer1_body,
        grid=(N // CH1,),
        in_specs=[
            _full((N, x.shape[1])), _full((x.shape[1], N)),
            _full(W1l.shape), _full((hid, x.shape[1])), _full((1, hid)),
            _full((hid, 1)), _full(W1r.shape), _full((1, hid)),
            _full(att1.shape), _full((hid, 1)), _full((hid, 1)),
            _full((1, hid)),
        ],
        out_specs=[pl.BlockSpec((CH1, hid), lambda i: (i, 0)),
                   pl.BlockSpec((hid, CH1), lambda i: (0, i))],
        out_shape=[jax.ShapeDtypeStruct((N, hid), f32),
                   jax.ShapeDtypeStruct((hid, N), f32)],
    )(x, x.T, W1l, W1l.T, b1l.reshape(1, -1), b1l.reshape(-1, 1), W1r,
      b1r.reshape(1, -1), att1, aval1, smask1, bias1.reshape(1, -1))

    wide = HEADS * OUT
    out = pl.pallas_call(
        _layer2_body,
        grid=(N // CH2,),
        in_specs=[
            _full((N, hid)), _full((hid, N)),
            _full(W2l.shape), _full((wide, hid)), _full((1, wide)),
            _full((wide, 1)), _full(W2r.shape), _full((1, wide)),
            _full(att2.shape), _full((wide, 1)), _full((wide, 1)),
            _full((1, OUT)),
        ],
        out_specs=pl.BlockSpec((CH2, OUT), lambda i: (i, 0)),
        out_shape=jax.ShapeDtypeStruct((N, OUT), f32),
    )(hfeat, hT, W2l, W2l.T, b2l.reshape(1, -1), b2l.reshape(-1, 1), W2r,
      b2r.reshape(1, -1), att2, aval2, smask2, bias2.reshape(1, -1))
    return out


# speedup vs baseline: 187.7062x; 1.0302x over previous
import jax
import jax.numpy as jnp
import numpy as np
from jax.experimental import pallas as pl
from jax.experimental.pallas import tpu as pltpu

N = 256
HEADS = 4
C1 = 32
OUT = 128
CH1 = 128
CH2 = 128
SIGNBIT = np.int32(-2147483648)
ABSMASK = np.int32(2147483647)


def _attend(xlT_s_h, xr_s_h, smask_h, xl_h, a06_col, logits_lin, f32):
    w = xlT_s_h[None, :, :] + xr_s_h[:, :, None]
    wi = jax.lax.bitcast_convert_type(w, jnp.int32)
    wi = jnp.bitwise_and(wi, ABSMASK)
    wi = jnp.bitwise_xor(wi, smask_h[None, :, :])
    ws = jax.lax.bitcast_convert_type(wi, f32)
    logits = logits_lin + jnp.sum(ws, axis=1)
    m = jnp.max(logits, axis=1, keepdims=True)
    e = jnp.exp(logits - m)
    alpha = e / jnp.sum(e, axis=1, keepdims=True)
    return jnp.dot(alpha, xl_h, preferred_element_type=f32)


def _layer_body(cdim, chunk, x_ref, xT_ref, Wl_ref, WlT_ref, blr_ref,
                blc_ref, Wr_ref, br_ref, att_ref, aval_ref, smask_ref,
                *out_and_scr):
    f32 = jnp.float32
    d0 = pl.program_id(0) * chunk
    x = x_ref[...]
    xl = jnp.dot(x, Wl_ref[...], preferred_element_type=f32) + blr_ref[...]
    xlT = (jnp.dot(WlT_ref[...], xT_ref[...], preferred_element_type=f32)
           + blc_ref[...])
    xc = x_ref[pl.ds(d0, chunk), :]
    xr = jnp.dot(xc, Wr_ref[...], preferred_element_type=f32) + br_ref[...]
    aval = aval_ref[...]
    smask = smask_ref[...]
    att = att_ref[...]
    xlT_s = xlT * aval
    xr_s = xr * aval[:, 0][None, :]
    cols = []
    for h in range(HEADS):
        sl = slice(h * cdim, (h + 1) * cdim)
        a06_col = 0.6 * att[h][:, None]
        A_row = jnp.sum(xlT[sl, :] * a06_col, axis=0, keepdims=True)
        B_col = jnp.sum(xr[:, sl] * a06_col[:, 0][None, :], axis=1,
                        keepdims=True)
        logits_lin = A_row + B_col
        cols.append(_attend(xlT_s[sl, :], xr_s[:, sl], smask[sl, :],
                            xl[:, sl], a06_col, logits_lin, f32))
    return cols, out_and_scr


def _layer1_body(x_ref, xT_ref, Wl_ref, WlT_ref, blr_ref, blc_ref, Wr_ref,
                 br_ref, att_ref, aval_ref, smask_ref, bias_ref, out_ref,
                 outT_ref):
    cols, _ = _layer_body(C1, CH1, x_ref, xT_ref, Wl_ref, WlT_ref, blr_ref,
                          blc_ref, Wr_ref, br_ref, att_ref, aval_ref,
                          smask_ref)
    hfeat = jnp.concatenate(cols, axis=1) + bias_ref[...]
    hfeat = jnp.maximum(hfeat, 0.0)
    out_ref[...] = hfeat
    outT_ref[...] = hfeat.T


def _layer2_body(x_ref, xT_ref, Wl_ref, WlT_ref, blr_ref, blc_ref, Wr_ref,
                 br_ref, att_ref, aval_ref, smask_ref, bias_ref, out_ref):
    cols, _ = _layer_body(OUT, CH2, x_ref, xT_ref, Wl_ref, WlT_ref, blr_ref,
                          blc_ref, Wr_ref, br_ref, att_ref, aval_ref,
                          smask_ref)
    acc = cols[0] + cols[1] + cols[2] + cols[3]
    out_ref[...] = acc * (1.0 / HEADS) + bias_ref[...]


def _full(shape):
    return pl.BlockSpec(shape, lambda i: (0,) * len(shape))


def _sign_inputs(att):
    a = att.reshape(-1, 1)
    aval = 0.4 * jnp.abs(a)
    smask = jnp.where(a < 0, SIGNBIT, jnp.int32(0))
    return aval, smask


def kernel(x, W1l, b1l, W1r, b1r, att1, bias1, W2l, b2l, W2r, b2r, att2,
           bias2):
    f32 = jnp.float32
    aval1, smask1 = _sign_inputs(att1)
    aval2, smask2 = _sign_inputs(att2)
    hid = HEADS * C1

    hfeat, hT = pl.pallas_call(
        _layer1_body,
        grid=(N // CH1,),
        in_specs=[
            _full((N, x.shape[1])), _full((x.shape[1], N)),
            _full(W1l.shape), _full((hid, x.shape[1])), _full((1, hid)),
            _full((hid, 1)), _full(W1r.shape), _full((1, hid)),
            _full(att1.shape), _full((hid, 1)), _full((hid, 1)),
            _full((1, hid)),
        ],
        out_specs=[pl.BlockSpec((CH1, hid), lambda i: (i, 0)),
                   pl.BlockSpec((hid, CH1), lambda i: (0, i))],
        out_shape=[jax.ShapeDtypeStruct((N, hid), f32),
                   jax.ShapeDtypeStruct((hid, N), f32)],
    )(x, x.T, W1l, W1l.T, b1l.reshape(1, -1), b1l.reshape(-1, 1), W1r,
      b1r.reshape(1, -1), att1, aval1, smask1, bias1.reshape(1, -1))

    wide = HEADS * OUT
    out = pl.pallas_call(
        _layer2_body,
        grid=(N // CH2,),
        in_specs=[
            _full((N, hid)), _full((hid, N)),
            _full(W2l.shape), _full((wide, hid)), _full((1, wide)),
            _full((wide, 1)), _full(W2r.shape), _full((1, wide)),
            _full(att2.shape), _full((wide, 1)), _full((wide, 1)),
            _full((1, OUT)),
        ],
        out_specs=pl.BlockSpec((CH2, OUT), lambda i: (i, 0)),
        out_shape=jax.ShapeDtypeStruct((N, OUT), f32),
    )(hfeat, hT, W2l, W2l.T, b2l.reshape(1, -1), b2l.reshape(-1, 1), W2r,
      b2r.reshape(1, -1), att2, aval2, smask2, bias2.reshape(1, -1))
    return out


# speedup vs baseline: 205.2386x; 1.0934x over previous
import jax
import jax.numpy as jnp
import numpy as np
from jax.experimental import pallas as pl
from jax.experimental.pallas import tpu as pltpu

N = 256
HEADS = 4
C1 = 32
OUT = 128
CH = 128
SIGNBIT = np.int32(-2147483648)
ABSMASK = np.int32(2147483647)


def _attend(xlT_s_h, xr_s_h, smask_h, xl_h, logits_lin, f32):
    w = xlT_s_h[None, :, :] + xr_s_h[:, :, None]
    wi = jax.lax.bitcast_convert_type(w, jnp.int32)
    wi = jnp.bitwise_and(wi, ABSMASK)
    wi = jnp.bitwise_xor(wi, smask_h[None, :, :])
    ws = jax.lax.bitcast_convert_type(wi, f32)
    logits = logits_lin + jnp.sum(ws, axis=1)
    m = jnp.max(logits, axis=1, keepdims=True)
    e = jnp.exp(logits - m)
    alpha = e / jnp.sum(e, axis=1, keepdims=True)
    return jnp.dot(alpha, xl_h, preferred_element_type=f32)


def _gat_layer(cdim, xl, xlT, xr, att, aval, smask):
    xlT_s = xlT * aval
    xr_s = xr * aval[:, 0][None, :]
    cols = []
    for h in range(HEADS):
        sl = slice(h * cdim, (h + 1) * cdim)
        a06_col = 0.6 * att[h][:, None]
        A_row = jnp.sum(xlT[sl, :] * a06_col, axis=0, keepdims=True)
        B_col = jnp.sum(xr[:, sl] * a06_col[:, 0][None, :], axis=1,
                        keepdims=True)
        cols.append(_attend(xlT_s[sl, :], xr_s[:, sl], smask[sl, :],
                            xl[:, sl], A_row + B_col, jnp.float32))
    return cols


def _gnn_body(x_ref, W1l_ref, b1l_ref, W1r_ref, b1r_ref, att1_ref,
              aval1_ref, smask1_ref, bias1_ref, W2l_ref, W2lT_ref, b2l_ref,
              b2lc_ref, W2r_ref, b2r_ref, att2_ref, aval2_ref, smask2_ref,
              bias2_ref, out_ref, h_scr, hT_scr):
    f32 = jnp.float32
    i = pl.program_id(0)
    nphase = pl.num_programs(0) // 2
    d0 = (i % nphase) * CH

    @pl.when(i < nphase)
    def _layer1():
        x = x_ref[...]
        xl = (jnp.dot(x, W1l_ref[...], preferred_element_type=f32)
              + b1l_ref[...])
        xlT = xl.T
        xc = x_ref[pl.ds(d0, CH), :]
        xr = (jnp.dot(xc, W1r_ref[...], preferred_element_type=f32)
              + b1r_ref[...])
        cols = _gat_layer(C1, xl, xlT, xr, att1_ref[...], aval1_ref[...],
                          smask1_ref[...])
        hfeat = jnp.concatenate(cols, axis=1) + bias1_ref[...]
        hfeat = jnp.maximum(hfeat, 0.0)
        h_scr[pl.ds(d0, CH), :] = hfeat
        hT_scr[:, pl.ds(d0, CH)] = hfeat.T

    @pl.when(i >= nphase)
    def _layer2():
        hfull = h_scr[...]
        xl = (jnp.dot(hfull, W2l_ref[...], preferred_element_type=f32)
              + b2l_ref[...])
        xlT = (jnp.dot(W2lT_ref[...], hT_scr[...],
                       preferred_element_type=f32) + b2lc_ref[...])
        hc = h_scr[pl.ds(d0, CH), :]
        xr = (jnp.dot(hc, W2r_ref[...], preferred_element_type=f32)
              + b2r_ref[...])
        cols = _gat_layer(OUT, xl, xlT, xr, att2_ref[...], aval2_ref[...],
                          smask2_ref[...])
        acc = cols[0] + cols[1] + cols[2] + cols[3]
        out_ref[...] = acc * (1.0 / HEADS) + bias2_ref[...]


def _full(shape):
    return pl.BlockSpec(shape, lambda i: (0,) * len(shape))


def _sign_inputs(att):
    a = att.reshape(-1, 1)
    aval = 0.4 * jnp.abs(a)
    smask = jnp.where(a < 0, SIGNBIT, np.int32(0))
    return aval, smask


def kernel(x, W1l, b1l, W1r, b1r, att1, bias1, W2l, b2l, W2r, b2r, att2,
           bias2):
    f32 = jnp.float32
    aval1, smask1 = _sign_inputs(att1)
    aval2, smask2 = _sign_inputs(att2)
    hid = HEADS * C1
    wide = HEADS * OUT
    nphase = N // CH

    return pl.pallas_call(
        _gnn_body,
        grid=(2 * nphase,),
        in_specs=[
            _full((N, x.shape[1])),
            _full(W1l.shape), _full((1, hid)), _full(W1r.shape),
            _full((1, hid)), _full(att1.shape), _full((hid, 1)),
            _full((hid, 1)), _full((1, hid)),
            _full(W2l.shape), _full((wide, hid)), _full((1, wide)),
            _full((wide, 1)), _full(W2r.shape), _full((1, wide)),
            _full(att2.shape), _full((wide, 1)), _full((wide, 1)),
            _full((1, OUT)),
        ],
        out_specs=pl.BlockSpec((CH, OUT),
                               lambda i: (jnp.maximum(i - nphase, 0), 0)),
        out_shape=jax.ShapeDtypeStruct((N, OUT), f32),
        scratch_shapes=[pltpu.VMEM((N, hid), f32),
                        pltpu.VMEM((hid, N), f32)],
    )(x, W1l, b1l.reshape(1, -1), W1r, b1r.reshape(1, -1), att1, aval1,
      smask1, bias1.reshape(1, -1), W2l, W2l.T, b2l.reshape(1, -1),
      b2l.reshape(-1, 1), W2r, b2r.reshape(1, -1), att2, aval2, smask2,
      bias2.reshape(1, -1))


# speedup vs baseline: 256.7966x; 1.2512x over previous
import jax
import jax.numpy as jnp
import numpy as np
from jax.experimental import pallas as pl
from jax.experimental.pallas import tpu as pltpu

N = 256
HEADS = 4
C1 = 32
OUT = 128
CH = 128
SIGNBIT16 = np.int16(-32768)
ABSMASK16 = np.int16(32767)


def _attend(xlT_s_h, xr_s_h, smask_h, xl_h, logits_lin, f32):
    w = xlT_s_h[None, :, :] + xr_s_h[:, :, None]
    wi = jax.lax.bitcast_convert_type(w, jnp.int16)
    wi = jnp.bitwise_and(wi, ABSMASK16)
    wi = jnp.bitwise_xor(wi, smask_h[None, :, :])
    ws = jax.lax.bitcast_convert_type(wi, jnp.bfloat16)
    logits = logits_lin + jnp.sum(ws.astype(f32), axis=1)
    m = jnp.max(logits, axis=1, keepdims=True)
    e = jnp.exp(logits - m)
    alpha = e / jnp.sum(e, axis=1, keepdims=True)
    return jnp.dot(alpha, xl_h, preferred_element_type=f32)


def _gat_layer(cdim, xl, xlT, xr, att, aval, smask):
    xlT_s = (xlT * aval).astype(jnp.bfloat16)
    xr_s = (xr * aval[:, 0][None, :]).astype(jnp.bfloat16)
    cols = []
    for h in range(HEADS):
        sl = slice(h * cdim, (h + 1) * cdim)
        a06_col = 0.6 * att[h][:, None]
        A_row = jnp.sum(xlT[sl, :] * a06_col, axis=0, keepdims=True)
        B_col = jnp.sum(xr[:, sl] * a06_col[:, 0][None, :], axis=1,
                        keepdims=True)
        cols.append(_attend(xlT_s[sl, :], xr_s[:, sl], smask[sl, :],
                            xl[:, sl], A_row + B_col, jnp.float32))
    return cols


def _gnn_body(x_ref, W1l_ref, b1l_ref, W1r_ref, b1r_ref, att1_ref,
              aval1_ref, smask1_ref, bias1_ref, W2l_ref, W2lT_ref, b2l_ref,
              b2lc_ref, W2r_ref, b2r_ref, att2_ref, aval2_ref, smask2_ref,
              bias2_ref, out_ref, h_scr, hT_scr):
    f32 = jnp.float32
    i = pl.program_id(0)
    nphase = pl.num_programs(0) // 2
    d0 = (i % nphase) * CH

    @pl.when(i < nphase)
    def _layer1():
        x = x_ref[...]
        xl = (jnp.dot(x, W1l_ref[...], preferred_element_type=f32)
              + b1l_ref[...])
        xlT = xl.T
        xc = x_ref[pl.ds(d0, CH), :]
        xr = (jnp.dot(xc, W1r_ref[...], preferred_element_type=f32)
              + b1r_ref[...])
        cols = _gat_layer(C1, xl, xlT, xr, att1_ref[...], aval1_ref[...],
                          smask1_ref[...])
        hfeat = jnp.concatenate(cols, axis=1) + bias1_ref[...]
        hfeat = jnp.maximum(hfeat, 0.0)
        h_scr[pl.ds(d0, CH), :] = hfeat
        hT_scr[:, pl.ds(d0, CH)] = hfeat.T

    @pl.when(i >= nphase)
    def _layer2():
        hfull = h_scr[...]
        xl = (jnp.dot(hfull, W2l_ref[...], preferred_element_type=f32)
              + b2l_ref[...])
        xlT = (jnp.dot(W2lT_ref[...], hT_scr[...],
                       preferred_element_type=f32) + b2lc_ref[...])
        hc = h_scr[pl.ds(d0, CH), :]
        xr = (jnp.dot(hc, W2r_ref[...], preferred_element_type=f32)
              + b2r_ref[...])
        cols = _gat_layer(OUT, xl, xlT, xr, att2_ref[...], aval2_ref[...],
                          smask2_ref[...])
        acc = cols[0] + cols[1] + cols[2] + cols[3]
        out_ref[...] = acc * (1.0 / HEADS) + bias2_ref[...]


def _full(shape):
    return pl.BlockSpec(shape, lambda i: (0,) * len(shape))


def _sign_inputs(att):
    a = att.reshape(-1, 1)
    aval = 0.4 * jnp.abs(a)
    smask = jnp.where(a < 0, SIGNBIT16, np.int16(0))
    return aval, smask


def kernel(x, W1l, b1l, W1r, b1r, att1, bias1, W2l, b2l, W2r, b2r, att2,
           bias2):
    f32 = jnp.float32
    aval1, smask1 = _sign_inputs(att1)
    aval2, smask2 = _sign_inputs(att2)
    hid = HEADS * C1
    wide = HEADS * OUT
    nphase = N // CH

    return pl.pallas_call(
        _gnn_body,
        grid=(2 * nphase,),
        in_specs=[
            _full((N, x.shape[1])),
            _full(W1l.shape), _full((1, hid)), _full(W1r.shape),
            _full((1, hid)), _full(att1.shape), _full((hid, 1)),
            _full((hid, 1)), _full((1, hid)),
            _full(W2l.shape), _full((wide, hid)), _full((1, wide)),
            _full((wide, 1)), _full(W2r.shape), _full((1, wide)),
            _full(att2.shape), _full((wide, 1)), _full((wide, 1)),
            _full((1, OUT)),
        ],
        out_specs=pl.BlockSpec((CH, OUT),
                               lambda i: (jnp.maximum(i - nphase, 0), 0)),
        out_shape=jax.ShapeDtypeStruct((N, OUT), f32),
        scratch_shapes=[pltpu.VMEM((N, hid), f32),
                        pltpu.VMEM((hid, N), f32)],
    )(x, W1l, b1l.reshape(1, -1), W1r, b1r.reshape(1, -1), att1, aval1,
      smask1, bias1.reshape(1, -1), W2l, W2l.T, b2l.reshape(1, -1),
      b2l.reshape(-1, 1), W2r, b2r.reshape(1, -1), att2, aval2, smask2,
      bias2.reshape(1, -1))


# speedup vs baseline: 416.9070x; 1.6235x over previous
import jax
import jax.numpy as jnp
import numpy as np
from jax.experimental import pallas as pl
from jax.experimental.pallas import tpu as pltpu

N = 256
HEADS = 4
C1 = 32
OUT = 128
CH = 128
SIGNBIT16 = np.int16(-32768)


def _attend(xlT_s_h, xr_s_h, smask_h, xl_h, logits_lin, f32):
    w = xlT_s_h[None, :, :] + xr_s_h[:, :, None]
    wi = jax.lax.bitcast_convert_type(w, jnp.int16)
    wi = jnp.bitwise_and(wi, np.int16(32767))
    wi = jnp.bitwise_xor(wi, smask_h[None, :, :])
    ws = jax.lax.bitcast_convert_type(wi, jnp.bfloat16)
    logits = logits_lin + jnp.sum(ws.astype(f32), axis=1)
    m = jnp.max(logits, axis=1, keepdims=True)
    e = jnp.exp(logits - m)
    alpha = e / jnp.sum(e, axis=1, keepdims=True)
    return jnp.dot(alpha, xl_h, preferred_element_type=f32)


def _gat_layer(cdim, xl, xlT, xr, att):
    bf16 = jnp.bfloat16
    cols = []
    for h in range(HEADS):
        sl = slice(h * cdim, (h + 1) * cdim)
        a_row = att[h][None, :]
        a_col = att[h][:, None]
        aval_row = 0.4 * jnp.abs(a_row)
        aval_col = 0.4 * jnp.abs(a_col)
        ai = jax.lax.bitcast_convert_type(a_col, jnp.int32)
        sm32 = jax.lax.shift_right_logical(
            jnp.bitwise_and(ai, np.int32(-2147483648)), 16)
        smask_h = sm32.astype(jnp.int16)
        A_row = 0.6 * jnp.sum(xlT[sl, :] * a_col, axis=0, keepdims=True)
        B_col = 0.6 * jnp.sum(xr[:, sl] * a_row, axis=1, keepdims=True)
        xlT_s = (xlT[sl, :] * aval_col).astype(bf16)
        xr_s = (xr[:, sl] * aval_row).astype(bf16)
        cols.append(_attend(xlT_s, xr_s, smask_h, xl[:, sl],
                            A_row + B_col, jnp.float32))
    return cols


def _gnn_body(x_ref, W1l_ref, b1l_ref, W1r_ref, b1r_ref, att1_ref,
              bias1_ref, W2l_ref, b2l_ref, W2r_ref, b2r_ref, att2_ref,
              bias2_ref, out_ref, h_scr, hT_scr):
    f32 = jnp.float32
    i = pl.program_id(0)
    nphase = pl.num_programs(0) // 2
    d0 = (i % nphase) * CH

    @pl.when(i < nphase)
    def _layer1():
        x = x_ref[...]
        xl = (jnp.dot(x, W1l_ref[...], preferred_element_type=f32)
              + b1l_ref[...])
        xlT = xl.T
        xc = x_ref[pl.ds(d0, CH), :]
        xr = (jnp.dot(xc, W1r_ref[...], preferred_element_type=f32)
              + b1r_ref[...])
        cols = _gat_layer(C1, xl, xlT, xr, att1_ref[...])
        hfeat = jnp.concatenate(cols, axis=1) + bias1_ref[...]
        hfeat = jnp.maximum(hfeat, 0.0)
        h_scr[pl.ds(d0, CH), :] = hfeat
        hT_scr[:, pl.ds(d0, CH)] = hfeat.T

    @pl.when(i >= nphase)
    def _layer2():
        hfull = h_scr[...]
        xl = (jnp.dot(hfull, W2l_ref[...], preferred_element_type=f32)
              + b2l_ref[...])
        xlT = jax.lax.dot_general(W2l_ref[...], hT_scr[...],
                                  (((0,), (0,)), ((), ())),
                                  preferred_element_type=f32)
        xlT = xlT + b2l_ref[...].T
        hc = h_scr[pl.ds(d0, CH), :]
        xr = (jnp.dot(hc, W2r_ref[...], preferred_element_type=f32)
              + b2r_ref[...])
        cols = _gat_layer(OUT, xl, xlT, xr, att2_ref[...])
        acc = cols[0] + cols[1] + cols[2] + cols[3]
        out_ref[...] = acc * (1.0 / HEADS) + bias2_ref[...]


def _full(shape):
    return pl.BlockSpec(shape, lambda i: (0,) * len(shape))


def kernel(x, W1l, b1l, W1r, b1r, att1, bias1, W2l, b2l, W2r, b2r, att2,
           bias2):
    f32 = jnp.float32
    hid = HEADS * C1
    wide = HEADS * OUT
    nphase = N // CH

    return pl.pallas_call(
        _gnn_body,
        grid=(2 * nphase,),
        in_specs=[
            _full((N, x.shape[1])),
            _full(W1l.shape), _full((1, hid)), _full(W1r.shape),
            _full((1, hid)), _full(att1.shape), _full((1, hid)),
            _full(W2l.shape), _full((1, wide)), _full(W2r.shape),
            _full((1, wide)), _full(att2.shape), _full((1, OUT)),
        ],
        out_specs=pl.BlockSpec((CH, OUT),
                               lambda i: (jnp.maximum(i - nphase, 0), 0)),
        out_shape=jax.ShapeDtypeStruct((N, OUT), f32),
        scratch_shapes=[pltpu.VMEM((N, hid), f32),
                        pltpu.VMEM((hid, N), f32)],
    )(x, W1l, b1l.reshape(1, -1), W1r, b1r.reshape(1, -1), att1,
      bias1.reshape(1, -1), W2l, b2l.reshape(1, -1), W2r,
      b2r.reshape(1, -1), att2, bias2.reshape(1, -1))


# speedup vs baseline: 505.1491x; 1.2117x over previous
import jax
import jax.numpy as jnp
import numpy as np
from jax.experimental import pallas as pl
from jax.experimental.pallas import tpu as pltpu

N = 256
HEADS = 4
C1 = 32
OUT = 128
CH = 128
SIGNBIT16 = np.int16(-32768)


def _attend(xlT_s_h, xr_s_h, smask_h, xl_h, logits_lin, f32):
    w = xlT_s_h[None, :, :] + xr_s_h[:, :, None]
    wi = jax.lax.bitcast_convert_type(w, jnp.int16)
    wi = jnp.bitwise_and(wi, np.int16(32767))
    wi = jnp.bitwise_xor(wi, smask_h[None, :, :])
    ws = jax.lax.bitcast_convert_type(wi, jnp.bfloat16)
    c = ws.shape[1]
    s1 = ws[:, :c // 2, :] + ws[:, c // 2:, :]
    s2 = s1[:, :c // 4, :] + s1[:, c // 4:, :]
    logits = logits_lin + jnp.sum(s2.astype(f32), axis=1)
    m = jnp.max(logits, axis=1, keepdims=True)
    e = jnp.exp(logits - m)
    alpha = e / jnp.sum(e, axis=1, keepdims=True)
    return jnp.dot(alpha, xl_h, preferred_element_type=f32)


def _gat_layer(cdim, xl, xlT, xr, att):
    bf16 = jnp.bfloat16
    cols = []
    for h in range(HEADS):
        sl = slice(h * cdim, (h + 1) * cdim)
        a_row = att[h][None, :]
        a_col = att[h][:, None]
        aval_row = 0.4 * jnp.abs(a_row)
        aval_col = 0.4 * jnp.abs(a_col)
        ai = jax.lax.bitcast_convert_type(a_col, jnp.int32)
        sm32 = jax.lax.shift_right_logical(
            jnp.bitwise_and(ai, np.int32(-2147483648)), 16)
        smask_h = sm32.astype(jnp.int16)
        A_row = 0.6 * jnp.sum(xlT[sl, :] * a_col, axis=0, keepdims=True)
        B_col = 0.6 * jnp.sum(xr[:, sl] * a_row, axis=1, keepdims=True)
        xlT_s = (xlT[sl, :] * aval_col).astype(bf16)
        xr_s = (xr[:, sl] * aval_row).astype(bf16)
        cols.append(_attend(xlT_s, xr_s, smask_h, xl[:, sl],
                            A_row + B_col, jnp.float32))
    return cols


def _gnn_body(x_ref, W1l_ref, b1l_ref, W1r_ref, b1r_ref, att1_ref,
              bias1_ref, W2l_ref, b2l_ref, W2r_ref, b2r_ref, att2_ref,
              bias2_ref, out_ref, h_scr, hT_scr):
    f32 = jnp.float32
    i = pl.program_id(0)
    nphase = pl.num_programs(0) // 2
    d0 = (i % nphase) * CH

    @pl.when(i < nphase)
    def _layer1():
        x = x_ref[...]
        xl = (jnp.dot(x, W1l_ref[...], preferred_element_type=f32)
              + b1l_ref[...])
        xlT = xl.T
        xc = x_ref[pl.ds(d0, CH), :]
        xr = (jnp.dot(xc, W1r_ref[...], preferred_element_type=f32)
              + b1r_ref[...])
        cols = _gat_layer(C1, xl, xlT, xr, att1_ref[...])
        hfeat = jnp.concatenate(cols, axis=1) + bias1_ref[...]
        hfeat = jnp.maximum(hfeat, 0.0)
        h_scr[pl.ds(d0, CH), :] = hfeat
        hT_scr[:, pl.ds(d0, CH)] = hfeat.T

    @pl.when(i >= nphase)
    def _layer2():
        hfull = h_scr[...]
        xl = (jnp.dot(hfull, W2l_ref[...], preferred_element_type=f32)
              + b2l_ref[...])
        xlT = jax.lax.dot_general(W2l_ref[...], hT_scr[...],
                                  (((0,), (0,)), ((), ())),
                                  preferred_element_type=f32)
        xlT = xlT + b2l_ref[...].T
        hc = h_scr[pl.ds(d0, CH), :]
        xr = (jnp.dot(hc, W2r_ref[...], preferred_element_type=f32)
              + b2r_ref[...])
        cols = _gat_layer(OUT, xl, xlT, xr, att2_ref[...])
        acc = cols[0] + cols[1] + cols[2] + cols[3]
        out_ref[...] = acc * (1.0 / HEADS) + bias2_ref[...]


def _full(shape):
    return pl.BlockSpec(shape, lambda i: (0,) * len(shape))


def kernel(x, W1l, b1l, W1r, b1r, att1, bias1, W2l, b2l, W2r, b2r, att2,
           bias2):
    f32 = jnp.float32
    hid = HEADS * C1
    wide = HEADS * OUT
    nphase = N // CH

    return pl.pallas_call(
        _gnn_body,
        grid=(2 * nphase,),
        in_specs=[
            _full((N, x.shape[1])),
            _full(W1l.shape), _full((1, hid)), _full(W1r.shape),
            _full((1, hid)), _full(att1.shape), _full((1, hid)),
            _full(W2l.shape), _full((1, wide)), _full(W2r.shape),
            _full((1, wide)), _full(att2.shape), _full((1, OUT)),
        ],
        out_specs=pl.BlockSpec((CH, OUT),
                               lambda i: (jnp.maximum(i - nphase, 0), 0)),
        out_shape=jax.ShapeDtypeStruct((N, OUT), f32),
        scratch_shapes=[pltpu.VMEM((N, hid), f32),
                        pltpu.VMEM((hid, N), f32)],
    )(x, W1l, b1l.reshape(1, -1), W1r, b1r.reshape(1, -1), att1,
      bias1.reshape(1, -1), W2l, b2l.reshape(1, -1), W2r,
      b2r.reshape(1, -1), att2, bias2.reshape(1, -1))
